# baseline (speedup 1.0000x reference)
import dataclasses
import functools

import jax
import jax.numpy as jnp
from jax import lax
from jax.experimental import pallas as pl
from jax.experimental.pallas import tpu as pltpu
from jax.experimental.pallas import tpu_sc as plsc

EMB = 64
LANES = 16
NUM_CORES = 2
NUM_SUBCORES = 16
NUM_WORKERS = NUM_CORES * NUM_SUBCORES
TPAD = 17


def _compiler_params():
    cp = pltpu.CompilerParams()
    fields = pltpu.CompilerParams.__dataclass_fields__
    if "needs_layout_passes" in fields:
        cp = dataclasses.replace(cp, needs_layout_passes=False)
    if "use_tc_tiling_on_sc" in fields:
        cp = dataclasses.replace(cp, use_tc_tiling_on_sc=False)
    return cp


def kernel(user_idx, item_idx, Gu, Gi):
    B = user_idx.shape[0]
    b_per_w = B // NUM_WORKERS
    groups = b_per_w // LANES
    chunks = EMB // LANES

    mesh = plsc.VectorSubcoreMesh(core_axis_name="c", subcore_axis_name="s")

    @functools.partial(
        pl.kernel,
        mesh=mesh,
        out_type=jax.ShapeDtypeStruct((B,), jnp.float32),
        scratch_types=[
            pltpu.VMEM((b_per_w,), jnp.int32),
            pltpu.VMEM((b_per_w,), jnp.int32),
            pltpu.VMEM((b_per_w, EMB), jnp.float32),
            pltpu.VMEM((b_per_w, EMB), jnp.float32),
            pltpu.VMEM((LANES, TPAD), jnp.float32),
            pltpu.VMEM((b_per_w,), jnp.float32),
            pltpu.SemaphoreType.DMA,
            pltpu.SemaphoreType.DMA,
        ],
        compiler_params=_compiler_params(),
    )
    def _k(uidx_hbm, iidx_hbm, gu_hbm, gi_hbm, out_hbm,
           uix_v, iix_v, gu_v, gi_v, tbuf, out_v, sem_u, sem_i):
        wid = lax.axis_index("s") * NUM_CORES + lax.axis_index("c")
        base = wid * b_per_w

        pltpu.sync_copy(uidx_hbm.at[pl.ds(base, b_per_w)], uix_v)
        pltpu.sync_copy(iidx_hbm.at[pl.ds(base, b_per_w)], iix_v)
        cp_u = pltpu.async_copy(gu_hbm.at[uix_v], gu_v, sem_u)
        cp_i = pltpu.async_copy(gi_hbm.at[iix_v], gi_v, sem_i)
        cp_u.wait()
        cp_i.wait()

        row_ids = lax.iota(jnp.int32, LANES)

        @pl.loop(0, groups)
        def _(g):
            row0 = g * LANES
            for r in range(LANES):
                row = row0 + r
                acc = gu_v[row, pl.ds(0, LANES)] * gi_v[row, pl.ds(0, LANES)]
                for c in range(1, chunks):
                    acc = acc + (gu_v[row, pl.ds(c * LANES, LANES)]
                                 * gi_v[row, pl.ds(c * LANES, LANES)])
                tbuf[r, pl.ds(0, LANES)] = acc
            o = jnp.zeros((LANES,), jnp.float32)
            for c in range(LANES):
                o = o + plsc.load_gather(
                    tbuf, [row_ids, jnp.full((LANES,), c, jnp.int32)])
            out_v[pl.ds(row0, LANES)] = o

        pltpu.sync_copy(out_v, out_hbm.at[pl.ds(base, b_per_w)])

    return _k(user_idx, item_idx, Gu, Gi)
